# Initial kernel scaffold; baseline (speedup 1.0000x reference)
import functools

import jax
import jax.numpy as jnp
from jax import lax
from jax.experimental import pallas as pl
from jax.experimental.pallas import tpu as pltpu
from jax.experimental.pallas import tpu_sc as plsc

_N = 20000
_NS = 16
_CPT = 1280
_NP = _NS * _CPT
_B = 256
_NB = _NP // _B
_BPT = _CPT // _B
_BV = _B // 16
_VB = _CPT // 16
_THR = 0.7
_EPS = 1e-9

_mesh = plsc.VectorSubcoreMesh(
    core_axis_name="c", subcore_axis_name="s", num_cores=1)


def _iou_hit(bx1, by1, bx2, by2, bar, x1, y1, x2, y2, ar):
    xx1 = jnp.maximum(bx1, x1)
    yy1 = jnp.maximum(by1, y1)
    xx2 = jnp.minimum(bx2, x2)
    yy2 = jnp.minimum(by2, y2)
    w = jnp.maximum(xx2 - xx1, 0.0)
    h = jnp.maximum(yy2 - yy1, 0.0)
    inter = w * h
    u = bar + ar - inter + _EPS
    return inter > _THR * u


def _nms_body(x1h, y1h, x2h, y2h, arh, flh, outh,
              cx1, cy1, cx2, cy2, car, cfl, kbuf, rbuf, kcnt, kshared):
    wid = lax.axis_index("s")
    base = wid * _CPT
    iota16 = lax.broadcasted_iota(jnp.int32, (16,), 0)

    pltpu.sync_copy(x1h.at[pl.ds(base, _CPT)], cx1)
    pltpu.sync_copy(y1h.at[pl.ds(base, _CPT)], cy1)
    pltpu.sync_copy(x2h.at[pl.ds(base, _CPT)], cx2)
    pltpu.sync_copy(y2h.at[pl.ds(base, _CPT)], cy2)
    pltpu.sync_copy(arh.at[pl.ds(base, _CPT)], car)
    pltpu.sync_copy(flh.at[pl.ds(base, _CPT)], cfl)

    def _apply(bb):
        pltpu.sync_copy(kshared.at[lax.rem(bb, 2)], rbuf)
        kcount = rbuf[5, 0].astype(jnp.int32)
        lstart = (bb + 1) * _B - base
        lv = jnp.clip(lstart // 16, 0, _VB)

        def body_k(k, _):
            bx1 = rbuf[0, k]
            by1 = rbuf[1, k]
            bx2 = rbuf[2, k]
            by2 = rbuf[3, k]
            bar = rbuf[4, k]

            def body_v(v, _2):
                oc = pl.multiple_of(v * 16, 16)
                sl = pl.ds(oc, 16)
                hit = _iou_hit(bx1, by1, bx2, by2, bar,
                               cx1[sl], cy1[sl], cx2[sl], cy2[sl], car[sl])
                cfl[sl] = cfl[sl] | hit.astype(jnp.int32)
                return 0

            lax.fori_loop(lv, _VB, body_v, 0)
            return 0

        lax.fori_loop(0, kcount, body_k, 0)

    def _resolve(b):
        o = lax.rem(b, _BPT) * _B
        kcnt[0] = 0

        def body_i(i, _):
            li = o + i

            @pl.when(cfl[li] == 0)
            def _kept():
                bx1 = cx1[li]
                by1 = cy1[li]
                bx2 = cx2[li]
                by2 = cy2[li]
                bar = car[li]
                kc = kcnt[0]
                kbuf[0, kc] = bx1
                kbuf[1, kc] = by1
                kbuf[2, kc] = bx2
                kbuf[3, kc] = by2
                kbuf[4, kc] = bar
                kcnt[0] = kc + 1

                def body_vv(vv, _2):
                    oc = pl.multiple_of(o + vv * 16, 16)
                    sl = pl.ds(oc, 16)
                    hit = _iou_hit(bx1, by1, bx2, by2, bar,
                                   cx1[sl], cy1[sl], cx2[sl], cy2[sl],
                                   car[sl])
                    hit = jnp.logical_and(hit, (oc + iota16) > li)
                    cfl[sl] = cfl[sl] | hit.astype(jnp.int32)
                    return 0

                lax.fori_loop(i // 16, _BV, body_vv, 0)

            return 0

        lax.fori_loop(0, _B, body_i, 0)
        kbuf[5, 0] = kcnt[0].astype(jnp.float32)
        pltpu.sync_copy(kbuf, kshared.at[lax.rem(b, 2)])

    def body_b(b, _):
        @pl.when(b > 0)
        def _():
            _apply(b - 1)

        @pl.when(wid == b // _BPT)
        def _():
            _resolve(b)

        plsc.subcore_barrier()
        return 0

    lax.fori_loop(0, _NB, body_b, 0)
    pltpu.sync_copy(cfl, outh.at[pl.ds(base, _CPT)])


_nms_call = functools.partial(
    pl.kernel,
    out_type=jax.ShapeDtypeStruct((_NP,), jnp.int32),
    mesh=_mesh,
    scratch_types=[
        pltpu.VMEM((_CPT,), jnp.float32),
        pltpu.VMEM((_CPT,), jnp.float32),
        pltpu.VMEM((_CPT,), jnp.float32),
        pltpu.VMEM((_CPT,), jnp.float32),
        pltpu.VMEM((_CPT,), jnp.float32),
        pltpu.VMEM((_CPT,), jnp.int32),
        pltpu.VMEM((6, _B), jnp.float32),
        pltpu.VMEM((6, _B), jnp.float32),
        pltpu.SMEM((1,), jnp.int32),
        pltpu.VMEM_SHARED((2, 6, _B), jnp.float32),
    ],
)(_nms_body)


@jax.jit
def kernel(boxes, scores):
    order = jnp.argsort(-scores)
    bs = boxes[order]
    x1 = bs[:, 0]
    y1 = bs[:, 1]
    x2 = bs[:, 2]
    y2 = bs[:, 3]
    ar = (x2 - x1) * (y2 - y1)
    npad = _NP - _N
    zpad = jnp.zeros((npad,), jnp.float32)
    opad = jnp.ones((npad,), jnp.float32)
    x1p = jnp.concatenate([x1, zpad])
    y1p = jnp.concatenate([y1, zpad])
    x2p = jnp.concatenate([x2, opad])
    y2p = jnp.concatenate([y2, opad])
    arp = jnp.concatenate([ar, opad])
    flg0 = jnp.concatenate(
        [jnp.zeros((_N,), jnp.int32), jnp.ones((npad,), jnp.int32)])
    outflg = _nms_call(x1p, y1p, x2p, y2p, arp, flg0)
    keep_sorted = outflg[:_N] == 0
    keep = jnp.zeros((_N,), jnp.bool_).at[order].set(keep_sorted)
    return scores * keep.astype(scores.dtype)


# speedup vs baseline: 7.2891x; 7.2891x over previous
import functools

import jax
import jax.numpy as jnp
from jax import lax
from jax.experimental import pallas as pl
from jax.experimental.pallas import tpu as pltpu
from jax.experimental.pallas import tpu_sc as plsc

_N = 20000
_NS = 16
_CPT = 1280
_NP = _NS * _CPT
_B = 256
_NB = _NP // _B
_BPT = _CPT // _B
_BV = _B // 16
_VB = _CPT // 16
_THR = 0.7
_EPS = 1e-9

_mesh = plsc.VectorSubcoreMesh(
    core_axis_name="c", subcore_axis_name="s", num_cores=1)


def _iou_hit(bx1, by1, bx2, by2, bar, x1, y1, x2, y2, ar):
    xx1 = jnp.maximum(bx1, x1)
    yy1 = jnp.maximum(by1, y1)
    xx2 = jnp.minimum(bx2, x2)
    yy2 = jnp.minimum(by2, y2)
    w = jnp.maximum(xx2 - xx1, 0.0)
    h = jnp.maximum(yy2 - yy1, 0.0)
    inter = w * h
    u = bar + ar - inter + _EPS
    return inter > _THR * u


def _nms_body(x1h, y1h, x2h, y2h, arh, flh, outh,
              cx1, cy1, cx2, cy2, car, cfl, kbuf, rbuf, kcnt, kshared):
    wid = lax.axis_index("s")
    base = wid * _CPT
    iota16 = lax.broadcasted_iota(jnp.int32, (16,), 0)

    pltpu.sync_copy(x1h.at[pl.ds(base, _CPT)], cx1)
    pltpu.sync_copy(y1h.at[pl.ds(base, _CPT)], cy1)
    pltpu.sync_copy(x2h.at[pl.ds(base, _CPT)], cx2)
    pltpu.sync_copy(y2h.at[pl.ds(base, _CPT)], cy2)
    pltpu.sync_copy(arh.at[pl.ds(base, _CPT)], car)
    pltpu.sync_copy(flh.at[pl.ds(base, _CPT)], cfl)

    def _bcast2(ref, r, cvec):
        return plsc.load_gather(ref, [cvec + (r * _B)])

    def _apply(bb):
        pltpu.sync_copy(kshared.at[lax.rem(bb, 2)], rbuf)
        zv = jnp.zeros((16,), jnp.int32)
        kcount = _bcast2(rbuf, 5, zv)[0].astype(jnp.int32)
        lstart = (bb + 1) * _B - base
        lv = jnp.clip(lstart // 16, 0, _VB)

        def body_k(k, _):
            kv = jnp.full((16,), k, jnp.int32)
            bx1 = _bcast2(rbuf, 0, kv)
            by1 = _bcast2(rbuf, 1, kv)
            bx2 = _bcast2(rbuf, 2, kv)
            by2 = _bcast2(rbuf, 3, kv)
            bar = _bcast2(rbuf, 4, kv)

            def body_v(v, _2):
                oc = pl.multiple_of(v * 16, 16)
                sl = pl.ds(oc, 16)
                hit = _iou_hit(bx1, by1, bx2, by2, bar,
                               cx1[sl], cy1[sl], cx2[sl], cy2[sl], car[sl])
                cfl[sl] = cfl[sl] | hit.astype(jnp.int32)
                return 0

            lax.fori_loop(lv, _VB, body_v, 0)
            return 0

        lax.fori_loop(0, kcount, body_k, 0)

    def _resolve(b):
        o = lax.rem(b, _BPT) * _B
        kcnt[0] = 0

        lane0 = iota16 == 0

        def body_i(i, _):
            li = o + i
            liv = jnp.full((16,), li, jnp.int32)
            fv = plsc.load_gather(cfl, [liv])

            @pl.when(fv[0] == 0)
            def _kept():
                bx1 = plsc.load_gather(cx1, [liv])
                by1 = plsc.load_gather(cy1, [liv])
                bx2 = plsc.load_gather(cx2, [liv])
                by2 = plsc.load_gather(cy2, [liv])
                bar = plsc.load_gather(car, [liv])
                kc = kcnt[0]
                kcv = jnp.full((16,), kc, jnp.int32)
                for r, val in ((0, bx1), (1, by1), (2, bx2), (3, by2),
                               (4, bar)):
                    plsc.store_scatter(kbuf, [kcv + (r * _B)], val,
                                       mask=lane0)
                kcnt[0] = kc + 1

                def body_vv(vv, _2):
                    oc = pl.multiple_of(o + vv * 16, 16)
                    sl = pl.ds(oc, 16)
                    hit = _iou_hit(bx1, by1, bx2, by2, bar,
                                   cx1[sl], cy1[sl], cx2[sl], cy2[sl],
                                   car[sl])
                    hit = jnp.logical_and(hit, (oc + iota16) > li)
                    cfl[sl] = cfl[sl] | hit.astype(jnp.int32)
                    return 0

                lax.fori_loop(i // 16, _BV, body_vv, 0)

            return 0

        lax.fori_loop(0, _B, body_i, 0)
        cntv = jnp.full((16,), kcnt[0].astype(jnp.float32))
        plsc.store_scatter(kbuf, [jnp.full((16,), 5 * _B, jnp.int32)],
                           cntv, mask=lane0)
        pltpu.sync_copy(kbuf, kshared.at[lax.rem(b, 2)])

    def body_b(b, _):
        @pl.when(b > 0)
        def _():
            _apply(b - 1)

        @pl.when(wid == b // _BPT)
        def _():
            _resolve(b)

        plsc.subcore_barrier()
        return 0

    lax.fori_loop(0, _NB, body_b, 0)
    pltpu.sync_copy(cfl, outh.at[pl.ds(base, _CPT)])


_nms_call = functools.partial(
    pl.kernel,
    out_type=jax.ShapeDtypeStruct((_NP,), jnp.int32),
    mesh=_mesh,
    compiler_params=pltpu.CompilerParams(needs_layout_passes=False),
    scratch_types=[
        pltpu.VMEM((_CPT,), jnp.float32),
        pltpu.VMEM((_CPT,), jnp.float32),
        pltpu.VMEM((_CPT,), jnp.float32),
        pltpu.VMEM((_CPT,), jnp.float32),
        pltpu.VMEM((_CPT,), jnp.float32),
        pltpu.VMEM((_CPT,), jnp.int32),
        pltpu.VMEM((6 * _B,), jnp.float32),
        pltpu.VMEM((6 * _B,), jnp.float32),
        pltpu.SMEM((1,), jnp.int32),
        pltpu.VMEM_SHARED((2, 6 * _B), jnp.float32),
    ],
)(_nms_body)


@jax.jit
def kernel(boxes, scores):
    order = jnp.argsort(-scores)
    bs = boxes[order]
    x1 = bs[:, 0]
    y1 = bs[:, 1]
    x2 = bs[:, 2]
    y2 = bs[:, 3]
    ar = (x2 - x1) * (y2 - y1)
    npad = _NP - _N
    zpad = jnp.zeros((npad,), jnp.float32)
    opad = jnp.ones((npad,), jnp.float32)
    x1p = jnp.concatenate([x1, zpad])
    y1p = jnp.concatenate([y1, zpad])
    x2p = jnp.concatenate([x2, opad])
    y2p = jnp.concatenate([y2, opad])
    arp = jnp.concatenate([ar, opad])
    flg0 = jnp.concatenate(
        [jnp.zeros((_N,), jnp.int32), jnp.ones((npad,), jnp.int32)])
    outflg = _nms_call(x1p, y1p, x2p, y2p, arp, flg0)
    keep_sorted = outflg[:_N] == 0
    keep = jnp.zeros((_N,), jnp.bool_).at[order].set(keep_sorted)
    return scores * keep.astype(scores.dtype)
